# Initial kernel scaffold; baseline (speedup 1.0000x reference)
import jax
import jax.numpy as jnp
from jax.experimental import pallas as pl


def kernel(x, value):
    raise NotImplementedError("write your pallas kernel here")


# speedup vs baseline: 1.0012x; 1.0012x over previous
import jax
import jax.numpy as jnp
from jax.experimental import pallas as pl


def _scale_body(v_ref, x_ref, o_ref):
    o_ref[...] = x_ref[...] * (1.0 / v_ref[0, 0])


def kernel(x, value):
    b, s, d = x.shape
    rows = b * s
    xf = x.reshape(rows, d)
    vf = value.reshape(1, 1)

    blk = 1024
    assert rows % blk == 0
    grid = rows // blk

    out = pl.pallas_call(
        _scale_body,
        grid=(grid,),
        in_specs=[
            pl.BlockSpec((1, 1), lambda i: (0, 0)),
            pl.BlockSpec((blk, d), lambda i: (i, 0)),
        ],
        out_specs=pl.BlockSpec((blk, d), lambda i: (i, 0)),
        out_shape=jax.ShapeDtypeStruct((rows, d), x.dtype),
    )(vf, xf)
    return out.reshape(b, s, d)
